# baseline (speedup 1.0000x reference)
import functools

import jax
import jax.numpy as jnp
from jax import lax
from jax.experimental import pallas as pl
from jax.experimental.pallas import tpu as pltpu
from jax.experimental.pallas import tpu_sc as plsc

_VOCAB = 1000000
_DIM = 64
_BATCH = 4096
_NUM_SAMPLED = 8192

_NC = 2
_NS = 16
_NW = _NC * _NS

_TRUE_PER_W = _BATCH // _NW
_SAMP_PER_W = _NUM_SAMPLED // _NW


def _sc_gather(weights, biases, true_classes, sampled):
  mesh = plsc.VectorSubcoreMesh(core_axis_name="c", subcore_axis_name="s")

  @functools.partial(
      pl.kernel,
      out_type=[
          jax.ShapeDtypeStruct((_BATCH, _DIM), jnp.float32),
          jax.ShapeDtypeStruct((_BATCH,), jnp.float32),
          jax.ShapeDtypeStruct((_NUM_SAMPLED, _DIM), jnp.float32),
          jax.ShapeDtypeStruct((_NUM_SAMPLED,), jnp.float32),
      ],
      mesh=mesh,
      compiler_params=pltpu.CompilerParams(use_tc_tiling_on_sc=False),
      scratch_types=[
          pltpu.VMEM((_TRUE_PER_W,), jnp.int32),
          pltpu.VMEM((_TRUE_PER_W, _DIM), jnp.float32),
          pltpu.VMEM((_TRUE_PER_W,), jnp.float32),
          pltpu.VMEM((_SAMP_PER_W,), jnp.int32),
          pltpu.VMEM((_SAMP_PER_W, _DIM), jnp.float32),
          pltpu.VMEM((_SAMP_PER_W,), jnp.float32),
          pltpu.SemaphoreType.DMA,
          pltpu.SemaphoreType.DMA,
          pltpu.SemaphoreType.DMA,
          pltpu.SemaphoreType.DMA,
      ],
  )
  def gather(w_hbm, b_hbm, tc_hbm, s_hbm,
             tw_out, tb_out, sw_out, sb_out,
             tidx_v, trow_v, tb_v, sidx_v, srow_v, sb_v,
             sem0, sem1, sem2, sem3):
    wid = lax.axis_index("s") * _NC + lax.axis_index("c")
    tbase = wid * _TRUE_PER_W
    sbase = wid * _SAMP_PER_W
    pltpu.sync_copy(tc_hbm.at[pl.ds(tbase, _TRUE_PER_W)], tidx_v)
    pltpu.sync_copy(s_hbm.at[pl.ds(sbase, _SAMP_PER_W)], sidx_v)
    c0 = pltpu.async_copy(w_hbm.at[tidx_v], trow_v, sem0)
    c1 = pltpu.async_copy(w_hbm.at[sidx_v], srow_v, sem1)
    c2 = pltpu.async_copy(b_hbm.at[tidx_v], tb_v, sem2)
    c3 = pltpu.async_copy(b_hbm.at[sidx_v], sb_v, sem3)
    c0.wait()
    c1.wait()
    c2.wait()
    c3.wait()
    pltpu.sync_copy(trow_v, tw_out.at[pl.ds(tbase, _TRUE_PER_W)])
    pltpu.sync_copy(tb_v, tb_out.at[pl.ds(tbase, _TRUE_PER_W)])
    pltpu.sync_copy(srow_v, sw_out.at[pl.ds(sbase, _SAMP_PER_W)])
    pltpu.sync_copy(sb_v, sb_out.at[pl.ds(sbase, _SAMP_PER_W)])

  return gather(weights, biases, true_classes, sampled)


_BB = 256
_SS = 2048


def _nce_body(x_ref, tw_ref, tb_ref, tc_ref, sw_ref, sb_ref, sid_ref,
              out_ref):
  j = pl.program_id(1)
  x = x_ref[...]
  w = sw_ref[...]
  logits = lax.dot_general(
      x.astype(jnp.bfloat16), w.astype(jnp.bfloat16),
      dimension_numbers=(((1,), (1,)), ((), ())),
      preferred_element_type=jnp.float32)
  sid = sid_ref[0, :].astype(jnp.float32)
  q = (jnp.log(sid + 2.0) - jnp.log(sid + 1.0)) / jnp.log(float(_VOCAB) + 1.0)
  corr = jnp.log(_NUM_SAMPLED * q + 1e-12)
  z = logits + (sb_ref[0, :] - corr)[None, :]
  sp = jnp.maximum(z, 0.0) + jnp.log1p(jnp.exp(-jnp.abs(z)))
  part = jnp.sum(sp, axis=1)

  @pl.when(j == 0)
  def _():
    tcid = tc_ref[0, :].astype(jnp.float32)
    qt = (jnp.log(tcid + 2.0) - jnp.log(tcid + 1.0)) / jnp.log(float(_VOCAB) + 1.0)
    tl = (jnp.sum(x * tw_ref[...], axis=1) + tb_ref[0, :]
          - jnp.log(_NUM_SAMPLED * qt + 1e-12))
    tz = -tl
    out_ref[0, :] = (jnp.maximum(tz, 0.0) + jnp.log1p(jnp.exp(-jnp.abs(tz)))
                     + part)

  @pl.when(j > 0)
  def _():
    out_ref[0, :] += part


def _tc_loss(inputs, true_w, true_b, true_classes, sampled_w, sampled_b,
             sampled):
  grid = (_BATCH // _BB, _NUM_SAMPLED // _SS)
  out = pl.pallas_call(
      _nce_body,
      grid=grid,
      in_specs=[
          pl.BlockSpec((_BB, _DIM), lambda i, j: (i, 0)),
          pl.BlockSpec((_BB, _DIM), lambda i, j: (i, 0)),
          pl.BlockSpec((1, _BB), lambda i, j: (0, i)),
          pl.BlockSpec((1, _BB), lambda i, j: (0, i)),
          pl.BlockSpec((_SS, _DIM), lambda i, j: (j, 0)),
          pl.BlockSpec((1, _SS), lambda i, j: (0, j)),
          pl.BlockSpec((1, _SS), lambda i, j: (0, j)),
      ],
      out_specs=pl.BlockSpec((1, _BB), lambda i, j: (0, i)),
      out_shape=jax.ShapeDtypeStruct((1, _BATCH), jnp.float32),
      compiler_params=pltpu.CompilerParams(
          dimension_semantics=("parallel", "arbitrary")),
  )(inputs, true_w, true_b.reshape(1, _BATCH),
    true_classes.reshape(1, _BATCH), sampled_w,
    sampled_b.reshape(1, _NUM_SAMPLED), sampled.reshape(1, _NUM_SAMPLED))
  return out.reshape(_BATCH)


def kernel(inputs, true_classes, sampled, weights, biases):
  true_w, true_b, sampled_w, sampled_b = _sc_gather(
      weights, biases, true_classes, sampled)
  return _tc_loss(inputs, true_w, true_b, true_classes, sampled_w,
                  sampled_b, sampled)


# speedup vs baseline: 1.0021x; 1.0021x over previous
import functools

import jax
import jax.numpy as jnp
from jax import lax
from jax.experimental import pallas as pl
from jax.experimental.pallas import tpu as pltpu
from jax.experimental.pallas import tpu_sc as plsc

_VOCAB = 1000000
_DIM = 64
_BATCH = 4096
_NUM_SAMPLED = 8192

_NC = 2
_NS = 16
_NW = _NC * _NS

_TRUE_PER_W = _BATCH // _NW
_SAMP_PER_W = _NUM_SAMPLED // _NW


def _sc_gather(weights, biases, true_classes, sampled):
  mesh = plsc.VectorSubcoreMesh(core_axis_name="c", subcore_axis_name="s")

  @functools.partial(
      pl.kernel,
      out_type=[
          jax.ShapeDtypeStruct((_BATCH, _DIM), jnp.float32),
          jax.ShapeDtypeStruct((_BATCH,), jnp.float32),
          jax.ShapeDtypeStruct((_NUM_SAMPLED, _DIM), jnp.float32),
          jax.ShapeDtypeStruct((_NUM_SAMPLED,), jnp.float32),
      ],
      mesh=mesh,
      compiler_params=pltpu.CompilerParams(
          use_tc_tiling_on_sc=False, skip_device_barrier=True),
      scratch_types=[
          pltpu.VMEM((_TRUE_PER_W,), jnp.int32),
          pltpu.VMEM((_TRUE_PER_W, _DIM), jnp.float32),
          pltpu.VMEM((_TRUE_PER_W,), jnp.float32),
          pltpu.VMEM((_SAMP_PER_W,), jnp.int32),
          pltpu.VMEM((_SAMP_PER_W, _DIM), jnp.float32),
          pltpu.VMEM((_SAMP_PER_W,), jnp.float32),
          pltpu.SemaphoreType.DMA,
          pltpu.SemaphoreType.DMA,
          pltpu.SemaphoreType.DMA,
          pltpu.SemaphoreType.DMA,
      ],
  )
  def gather(w_hbm, b_hbm, tc_hbm, s_hbm,
             tw_out, tb_out, sw_out, sb_out,
             tidx_v, trow_v, tb_v, sidx_v, srow_v, sb_v,
             sem0, sem1, sem2, sem3):
    wid = lax.axis_index("s") * _NC + lax.axis_index("c")
    tbase = wid * _TRUE_PER_W
    sbase = wid * _SAMP_PER_W
    pltpu.sync_copy(tc_hbm.at[pl.ds(tbase, _TRUE_PER_W)], tidx_v)
    pltpu.sync_copy(s_hbm.at[pl.ds(sbase, _SAMP_PER_W)], sidx_v)
    c0 = pltpu.async_copy(w_hbm.at[tidx_v], trow_v, sem0)
    c1 = pltpu.async_copy(w_hbm.at[sidx_v], srow_v, sem1)
    c2 = pltpu.async_copy(b_hbm.at[tidx_v], tb_v, sem2)
    c3 = pltpu.async_copy(b_hbm.at[sidx_v], sb_v, sem3)
    c0.wait()
    c1.wait()
    c2.wait()
    c3.wait()
    pltpu.sync_copy(trow_v, tw_out.at[pl.ds(tbase, _TRUE_PER_W)])
    pltpu.sync_copy(tb_v, tb_out.at[pl.ds(tbase, _TRUE_PER_W)])
    pltpu.sync_copy(srow_v, sw_out.at[pl.ds(sbase, _SAMP_PER_W)])
    pltpu.sync_copy(sb_v, sb_out.at[pl.ds(sbase, _SAMP_PER_W)])

  return gather(weights, biases, true_classes, sampled)


_BB = 256
_SS = 2048


def _nce_body(x_ref, tw_ref, tb_ref, tc_ref, sw_ref, sb_ref, sid_ref,
              out_ref):
  j = pl.program_id(1)
  x = x_ref[...]
  w = sw_ref[...]
  logits = lax.dot_general(
      x.astype(jnp.bfloat16), w.astype(jnp.bfloat16),
      dimension_numbers=(((1,), (1,)), ((), ())),
      preferred_element_type=jnp.float32)
  sid = sid_ref[0, :].astype(jnp.float32)
  q = (jnp.log(sid + 2.0) - jnp.log(sid + 1.0)) / jnp.log(float(_VOCAB) + 1.0)
  corr = jnp.log(_NUM_SAMPLED * q + 1e-12)
  z = logits + (sb_ref[0, :] - corr)[None, :]
  sp = jnp.maximum(z, 0.0) + jnp.log1p(jnp.exp(-jnp.abs(z)))
  part = jnp.sum(sp, axis=1)

  @pl.when(j == 0)
  def _():
    tcid = tc_ref[0, :].astype(jnp.float32)
    qt = (jnp.log(tcid + 2.0) - jnp.log(tcid + 1.0)) / jnp.log(float(_VOCAB) + 1.0)
    tl = (jnp.sum(x * tw_ref[...], axis=1) + tb_ref[0, :]
          - jnp.log(_NUM_SAMPLED * qt + 1e-12))
    tz = -tl
    out_ref[0, :] = (jnp.maximum(tz, 0.0) + jnp.log1p(jnp.exp(-jnp.abs(tz)))
                     + part)

  @pl.when(j > 0)
  def _():
    out_ref[0, :] += part


def _tc_loss(inputs, true_w, true_b, true_classes, sampled_w, sampled_b,
             sampled):
  grid = (_BATCH // _BB, _NUM_SAMPLED // _SS)
  out = pl.pallas_call(
      _nce_body,
      grid=grid,
      in_specs=[
          pl.BlockSpec((_BB, _DIM), lambda i, j: (i, 0)),
          pl.BlockSpec((_BB, _DIM), lambda i, j: (i, 0)),
          pl.BlockSpec((1, _BB), lambda i, j: (0, i)),
          pl.BlockSpec((1, _BB), lambda i, j: (0, i)),
          pl.BlockSpec((_SS, _DIM), lambda i, j: (j, 0)),
          pl.BlockSpec((1, _SS), lambda i, j: (0, j)),
          pl.BlockSpec((1, _SS), lambda i, j: (0, j)),
      ],
      out_specs=pl.BlockSpec((1, _BB), lambda i, j: (0, i)),
      out_shape=jax.ShapeDtypeStruct((1, _BATCH), jnp.float32),
      compiler_params=pltpu.CompilerParams(
          dimension_semantics=("parallel", "arbitrary")),
  )(inputs, true_w, true_b.reshape(1, _BATCH),
    true_classes.reshape(1, _BATCH), sampled_w,
    sampled_b.reshape(1, _NUM_SAMPLED), sampled.reshape(1, _NUM_SAMPLED))
  return out.reshape(_BATCH)


def kernel(inputs, true_classes, sampled, weights, biases):
  true_w, true_b, sampled_w, sampled_b = _sc_gather(
      weights, biases, true_classes, sampled)
  return _tc_loss(inputs, true_w, true_b, true_classes, sampled_w,
                  sampled_b, sampled)
